# baseline (speedup 1.0000x reference)
import numpy as np
import jax
import jax.numpy as jnp
from jax import lax
from jax.experimental import pallas as pl
from jax.experimental.pallas import tpu as pltpu
from jax.experimental.pallas import tpu_sc as plsc

VOCAB = 32000
D_MODEL = 2048
MAX_SEQ = 2048
PE_BASE = 10000.0

B = 4
S = 2048
NC = 2
NS = 16
NW = NC * NS
POS_PER_W = S // NW
K = 4
NCHUNK = POS_PER_W // K
LANES = 16
VECS_PER_ROW = D_MODEL // LANES
NBUF = 3


def _positional_encoding():
    pos = np.arange(MAX_SEQ, dtype=np.float32)[:, None]
    i = np.arange(0, D_MODEL, 2, dtype=np.float32)
    div = np.power(np.float32(PE_BASE), i / np.float32(D_MODEL))
    ang = (pos / div).astype(np.float32)
    pe = np.zeros((MAX_SEQ, D_MODEL), dtype=np.float32)
    pe[:, 0::2] = np.sin(ang)
    pe[:, 1::2] = np.cos(ang)
    return pe


def _pe_packed_i32():
    import ml_dtypes
    pe = _positional_encoding()
    bits = pe.astype(ml_dtypes.bfloat16).view(np.uint16).astype(np.uint32)
    blk = bits.reshape(MAX_SEQ, D_MODEL // 32, 2, 16)
    packed = blk[:, :, 0, :] | (blk[:, :, 1, :] << 16)
    return np.ascontiguousarray(packed.reshape(MAX_SEQ, D_MODEL // 2)).view(np.int32)


_PE_NP = _pe_packed_i32()


def _sc_body(tokens_hbm, pe_hbm, table_hbm, out_hbm, *scratch):
    rows = [[scratch[b * NBUF + s] for s in range(NBUF)] for b in range(B)]
    pe_v = list(scratch[B * NBUF:B * NBUF + NBUF])
    idx_all = scratch[B * NBUF + NBUF]
    gsem = list(scratch[B * NBUF + NBUF + 1:B * NBUF + NBUF + 1 + NBUF])
    osem = list(scratch[B * NBUF + NBUF + 1 + NBUF:])

    wid = lax.axis_index("s") * NC + lax.axis_index("c")
    pos0 = wid * POS_PER_W

    for b in range(B):
        pltpu.sync_copy(tokens_hbm.at[b, pl.ds(pos0, POS_PER_W)],
                        idx_all.at[b])

    def start_unit(c):
        s = c % NBUF
        h = [pltpu.async_copy(pe_hbm.at[pl.ds(pos0 + c * K, K)],
                              pe_v[s], gsem[s])]
        for b in range(B):
            h.append(pltpu.async_copy(
                table_hbm.at[idx_all.at[b, pl.ds(c * K, K)]],
                rows[b][s], gsem[s]))
        return h

    def start_out(c):
        s = c % NBUF
        return [pltpu.async_copy(rows[b][s],
                                 out_hbm.at[b, pl.ds(pos0 + c * K, K)],
                                 osem[s])
                for b in range(B)]

    g_h = {0: start_unit(0)}
    o_h = {}

    for c in range(NCHUNK):
        s = c % NBUF
        nxt = c + 1
        if nxt < NCHUNK:
            prev = nxt - NBUF
            if prev >= 0:
                for h in o_h[prev]:
                    h.wait()
            g_h[nxt] = start_unit(nxt)
        for h in g_h[c]:
            h.wait()

        pe_s = pe_v[s]
        row_s = [rows[b][s] for b in range(B)]

        for r in range(K):
            def add_body(j, _, r=r):
                col = j * 2 * LANES
                w = pe_s[r, pl.ds(j * LANES, LANES)]
                lo = lax.bitcast_convert_type(w << 16, jnp.float32)
                hi = lax.bitcast_convert_type(w & jnp.int32(-65536), jnp.float32)
                for b in range(B):
                    row_s[b][r, pl.ds(col, LANES)] = (
                        row_s[b][r, pl.ds(col, LANES)] + lo
                    )
                    row_s[b][r, pl.ds(col + LANES, LANES)] = (
                        row_s[b][r, pl.ds(col + LANES, LANES)] + hi
                    )
                return 0

            lax.fori_loop(0, VECS_PER_ROW // 2, add_body, 0)

        o_h[c] = start_out(c)

    for c in range(max(0, NCHUNK - NBUF), NCHUNK):
        if c in o_h:
            for h in o_h[c]:
                h.wait()


@jax.jit
def _run(tokens, embedding_table):
    pe = jnp.asarray(_PE_NP)
    mesh = plsc.VectorSubcoreMesh(
        core_axis_name="c", subcore_axis_name="s", num_cores=NC, num_subcores=NS
    )
    scratch = (
        [pltpu.VMEM((K, D_MODEL), jnp.float32) for _ in range(B * NBUF)]
        + [pltpu.VMEM((K, D_MODEL // 2), jnp.int32) for _ in range(NBUF)]
        + [pltpu.VMEM((B, POS_PER_W), jnp.int32)]
        + [pltpu.SemaphoreType.DMA for _ in range(2 * NBUF)]
    )
    f = pl.kernel(
        _sc_body,
        out_type=jax.ShapeDtypeStruct((B, S, D_MODEL), jnp.float32),
        mesh=mesh,
        scratch_types=scratch,
    )
    return f(tokens, pe, embedding_table)


def kernel(tokens, embedding_table):
    return _run(tokens, embedding_table)


# speedup vs baseline: 1.1482x; 1.1482x over previous
import numpy as np
import jax
import jax.numpy as jnp
from jax import lax
from jax.experimental import pallas as pl
from jax.experimental.pallas import tpu as pltpu
from jax.experimental.pallas import tpu_sc as plsc

VOCAB = 32000
D_MODEL = 2048
MAX_SEQ = 2048
PE_BASE = 10000.0

B = 4
S = 2048
NC = 2
NS = 16
NW = NC * NS
POS_PER_W = S // NW
K = 4
NCHUNK = POS_PER_W // K
LANES = 16
VECS_PER_ROW = D_MODEL // LANES
NBUF = 3


def _positional_encoding():
    pos = np.arange(MAX_SEQ, dtype=np.float32)[:, None]
    i = np.arange(0, D_MODEL, 2, dtype=np.float32)
    div = np.power(np.float32(PE_BASE), i / np.float32(D_MODEL))
    ang = (pos / div).astype(np.float32)
    pe = np.zeros((MAX_SEQ, D_MODEL), dtype=np.float32)
    pe[:, 0::2] = np.sin(ang)
    pe[:, 1::2] = np.cos(ang)
    return pe


_PE_NP = _positional_encoding()


def _sc_body(tokens_hbm, pe_hbm, table_hbm, out_hbm, *scratch):
    rows = [[scratch[b * NBUF + s] for s in range(NBUF)] for b in range(B)]
    pe_v = list(scratch[B * NBUF:B * NBUF + NBUF])
    idx_all = scratch[B * NBUF + NBUF]
    gsem = list(scratch[B * NBUF + NBUF + 1:B * NBUF + NBUF + 1 + NBUF])
    osem = list(scratch[B * NBUF + NBUF + 1 + NBUF:])

    wid = lax.axis_index("s") * NC + lax.axis_index("c")
    pos0 = wid * POS_PER_W

    for b in range(B):
        pltpu.sync_copy(tokens_hbm.at[b, pl.ds(pos0, POS_PER_W)],
                        idx_all.at[b])

    def start_unit(c):
        s = c % NBUF
        h = [pltpu.async_copy(pe_hbm.at[pl.ds(pos0 + c * K, K)],
                              pe_v[s], gsem[s])]
        for b in range(B):
            h.append(pltpu.async_copy(
                table_hbm.at[idx_all.at[b, pl.ds(c * K, K)]],
                rows[b][s], gsem[s]))
        return h

    def start_out(c):
        s = c % NBUF
        return [pltpu.async_copy(rows[b][s],
                                 out_hbm.at[b, pl.ds(pos0 + c * K, K)],
                                 osem[s])
                for b in range(B)]

    g_h = {0: start_unit(0)}
    o_h = {}

    for c in range(NCHUNK):
        s = c % NBUF
        nxt = c + 1
        if nxt < NCHUNK:
            prev = nxt - NBUF
            if prev >= 0:
                for h in o_h[prev]:
                    h.wait()
            g_h[nxt] = start_unit(nxt)
        for h in g_h[c]:
            h.wait()

        pe_s = pe_v[s]
        row_s = [rows[b][s] for b in range(B)]

        for r in range(K):
            @plsc.parallel_loop(0, D_MODEL, LANES, unroll=4)
            def add_body(col, r=r):
                pv = pe_s[r, pl.ds(col, LANES)]
                for b in range(B):
                    row_s[b][r, pl.ds(col, LANES)] = (
                        row_s[b][r, pl.ds(col, LANES)] + pv
                    )

        o_h[c] = start_out(c)

    for c in range(max(0, NCHUNK - NBUF), NCHUNK):
        if c in o_h:
            for h in o_h[c]:
                h.wait()


@jax.jit
def _run(tokens, embedding_table):
    pe = jnp.asarray(_PE_NP)
    mesh = plsc.VectorSubcoreMesh(
        core_axis_name="c", subcore_axis_name="s", num_cores=NC, num_subcores=NS
    )
    scratch = (
        [pltpu.VMEM((K, D_MODEL), jnp.float32) for _ in range(B * NBUF)]
        + [pltpu.VMEM((K, D_MODEL), jnp.float32) for _ in range(NBUF)]
        + [pltpu.VMEM((B, POS_PER_W), jnp.int32)]
        + [pltpu.SemaphoreType.DMA for _ in range(2 * NBUF)]
    )
    f = pl.kernel(
        _sc_body,
        out_type=jax.ShapeDtypeStruct((B, S, D_MODEL), jnp.float32),
        mesh=mesh,
        scratch_types=scratch,
    )
    return f(tokens, pe, embedding_table)


def kernel(tokens, embedding_table):
    return _run(tokens, embedding_table)


# speedup vs baseline: 1.2108x; 1.0546x over previous
import numpy as np
import jax
import jax.numpy as jnp
from jax import lax
from jax.experimental import pallas as pl
from jax.experimental.pallas import tpu as pltpu
from jax.experimental.pallas import tpu_sc as plsc

VOCAB = 32000
D_MODEL = 2048
MAX_SEQ = 2048
PE_BASE = 10000.0

B = 4
S = 2048
NC = 2
NS = 16
NW = NC * NS
POS_PER_W = S // NW
K = 4
NCHUNK = POS_PER_W // K
LANES = 16
VECS_PER_ROW = D_MODEL // LANES
NBUF = 3


def _positional_encoding():
    pos = np.arange(MAX_SEQ, dtype=np.float32)[:, None]
    i = np.arange(0, D_MODEL, 2, dtype=np.float32)
    div = np.power(np.float32(PE_BASE), i / np.float32(D_MODEL))
    ang = (pos / div).astype(np.float32)
    pe = np.zeros((MAX_SEQ, D_MODEL), dtype=np.float32)
    pe[:, 0::2] = np.sin(ang)
    pe[:, 1::2] = np.cos(ang)
    return pe


def _pe_packed_i32():
    import ml_dtypes
    pe = _positional_encoding()
    bits = pe.astype(ml_dtypes.bfloat16).view(np.uint16).astype(np.uint32)
    blk = bits.reshape(MAX_SEQ, D_MODEL // 32, 2, 16)
    packed = blk[:, :, 0, :] | (blk[:, :, 1, :] << 16)
    return np.ascontiguousarray(
        packed.reshape(MAX_SEQ, D_MODEL // 2).view(np.int32))


_PE_NP = _pe_packed_i32()


def _sc_body(tokens_hbm, pe_hbm, table_hbm, out_hbm, *scratch):
    rows = [[scratch[b * NBUF + s] for s in range(NBUF)] for b in range(B)]
    pe_v = list(scratch[B * NBUF:B * NBUF + NBUF])
    idx_all = scratch[B * NBUF + NBUF]
    gsem = list(scratch[B * NBUF + NBUF + 1:B * NBUF + NBUF + 1 + NBUF])
    osem = list(scratch[B * NBUF + NBUF + 1 + NBUF:])

    wid = lax.axis_index("s") * NC + lax.axis_index("c")
    pos0 = wid * POS_PER_W

    for b in range(B):
        pltpu.sync_copy(tokens_hbm.at[b, pl.ds(pos0, POS_PER_W)],
                        idx_all.at[b])

    def start_unit(c):
        s = c % NBUF
        h = [pltpu.async_copy(pe_hbm.at[pl.ds(pos0 + c * K, K)],
                              pe_v[s], gsem[s])]
        for b in range(B):
            h.append(pltpu.async_copy(
                table_hbm.at[idx_all.at[b, pl.ds(c * K, K)]],
                rows[b][s], gsem[s]))
        return h

    def start_out(c):
        s = c % NBUF
        return [pltpu.async_copy(rows[b][s],
                                 out_hbm.at[b, pl.ds(pos0 + c * K, K)],
                                 osem[s])
                for b in range(B)]

    g_h = {0: start_unit(0)}
    o_h = {}

    for c in range(NCHUNK):
        s = c % NBUF
        nxt = c + 1
        if nxt < NCHUNK:
            prev = nxt - NBUF
            if prev >= 0:
                for h in o_h[prev]:
                    h.wait()
            g_h[nxt] = start_unit(nxt)
        for h in g_h[c]:
            h.wait()

        pe_s = pe_v[s]
        row_s = [rows[b][s] for b in range(B)]

        for r in range(K):
            @plsc.parallel_loop(0, D_MODEL // 2, LANES, unroll=2)
            def add_body(half, r=r):
                col = half * 2
                w = pe_s[r, pl.ds(half, LANES)]
                lo = lax.bitcast_convert_type(w << 16, jnp.float32)
                hi = lax.bitcast_convert_type(w & jnp.int32(-65536),
                                              jnp.float32)
                for b in range(B):
                    row_s[b][r, pl.ds(col, LANES)] = (
                        row_s[b][r, pl.ds(col, LANES)] + lo
                    )
                    row_s[b][r, pl.ds(col + LANES, LANES)] = (
                        row_s[b][r, pl.ds(col + LANES, LANES)] + hi
                    )

        o_h[c] = start_out(c)

    for c in range(max(0, NCHUNK - NBUF), NCHUNK):
        if c in o_h:
            for h in o_h[c]:
                h.wait()


@jax.jit
def _run(tokens, embedding_table):
    pe = jnp.asarray(_PE_NP)
    mesh = plsc.VectorSubcoreMesh(
        core_axis_name="c", subcore_axis_name="s", num_cores=NC, num_subcores=NS
    )
    scratch = (
        [pltpu.VMEM((K, D_MODEL), jnp.float32) for _ in range(B * NBUF)]
        + [pltpu.VMEM((K, D_MODEL // 2), jnp.int32) for _ in range(NBUF)]
        + [pltpu.VMEM((B, POS_PER_W), jnp.int32)]
        + [pltpu.SemaphoreType.DMA for _ in range(2 * NBUF)]
    )
    f = pl.kernel(
        _sc_body,
        out_type=jax.ShapeDtypeStruct((B, S, D_MODEL), jnp.float32),
        mesh=mesh,
        scratch_types=scratch,
    )
    return f(tokens, pe, embedding_table)


def kernel(tokens, embedding_table):
    return _run(tokens, embedding_table)


# speedup vs baseline: 1.2958x; 1.0702x over previous
import numpy as np
import jax
import jax.numpy as jnp
from jax import lax
from jax.experimental import pallas as pl
from jax.experimental.pallas import tpu as pltpu
from jax.experimental.pallas import tpu_sc as plsc

VOCAB = 32000
D_MODEL = 2048
MAX_SEQ = 2048
PE_BASE = 10000.0

B = 4
S = 2048
NC = 2
NS = 16
NW = NC * NS
POS_PER_W = S // NW
K = 4
NCHUNK = POS_PER_W // K
LANES = 16
VECS_PER_ROW = D_MODEL // LANES
NBUF = 3


def _positional_encoding():
    pos = np.arange(MAX_SEQ, dtype=np.float32)[:, None]
    i = np.arange(0, D_MODEL, 2, dtype=np.float32)
    div = np.power(np.float32(PE_BASE), i / np.float32(D_MODEL))
    ang = (pos / div).astype(np.float32)
    pe = np.zeros((MAX_SEQ, D_MODEL), dtype=np.float32)
    pe[:, 0::2] = np.sin(ang)
    pe[:, 1::2] = np.cos(ang)
    return pe


def _pe_packed_i32():
    import ml_dtypes
    pe = _positional_encoding()
    bits = pe.astype(ml_dtypes.bfloat16).view(np.uint16).astype(np.uint32)
    blk = bits.reshape(MAX_SEQ, D_MODEL // 32, 2, 16)
    packed = blk[:, :, 0, :] | (blk[:, :, 1, :] << 16)
    return np.ascontiguousarray(
        packed.reshape(MAX_SEQ, D_MODEL // 2).view(np.int32))


_PE_NP = _pe_packed_i32()


def _sc_body(tokens_hbm, pe_hbm, table_hbm, out_hbm, *scratch):
    rows = [[scratch[b * NBUF + s] for s in range(NBUF)] for b in range(B)]
    pe_v = list(scratch[B * NBUF:B * NBUF + NBUF])
    idx_all = scratch[B * NBUF + NBUF]
    gsem = list(scratch[B * NBUF + NBUF + 1:B * NBUF + NBUF + 1 + NBUF])
    osem = list(scratch[B * NBUF + NBUF + 1 + NBUF:])

    wid = lax.axis_index("s") * NC + lax.axis_index("c")
    pos0 = wid * POS_PER_W

    for b in range(B):
        pltpu.sync_copy(tokens_hbm.at[b, pl.ds(pos0, POS_PER_W)],
                        idx_all.at[b])

    def start_unit(c):
        s = c % NBUF
        h = [pltpu.async_copy(pe_hbm.at[pl.ds(pos0 + c * K, K)],
                              pe_v[s], gsem[s])]
        for b in range(B):
            h.append(pltpu.async_copy(
                table_hbm.at[idx_all.at[b, pl.ds(c * K, K)]],
                rows[b][s], gsem[s]))
        return h

    def start_out(c):
        s = c % NBUF
        return [pltpu.async_copy(rows[b][s],
                                 out_hbm.at[b, pl.ds(pos0 + c * K, K)],
                                 osem[s])
                for b in range(B)]

    g_h = {0: start_unit(0)}
    o_h = {}

    for c in range(NCHUNK):
        s = c % NBUF
        nxt = c + 1
        if nxt < NCHUNK:
            prev = nxt - NBUF
            if prev >= 0:
                for h in o_h[prev]:
                    h.wait()
            g_h[nxt] = start_unit(nxt)
        for h in g_h[c]:
            h.wait()

        pe_s = pe_v[s]
        row_s = [rows[b][s] for b in range(B)]

        HALVES = D_MODEL // (2 * LANES)

        @plsc.parallel_loop(0, K * HALVES, 1, unroll=4)
        def add_body(o):
            r = o // HALVES
            half = (o % HALVES) * LANES
            col = half * 2
            w = pe_s[r, pl.ds(half, LANES)]
            lo = lax.bitcast_convert_type(w << 16, jnp.float32)
            hi = lax.bitcast_convert_type(w & jnp.int32(-65536), jnp.float32)
            for b in range(B):
                row_s[b][r, pl.ds(col, LANES)] = (
                    row_s[b][r, pl.ds(col, LANES)] + lo
                )
                row_s[b][r, pl.ds(col + LANES, LANES)] = (
                    row_s[b][r, pl.ds(col + LANES, LANES)] + hi
                )

        o_h[c] = start_out(c)

    for c in range(max(0, NCHUNK - NBUF), NCHUNK):
        if c in o_h:
            for h in o_h[c]:
                h.wait()


@jax.jit
def _run(tokens, embedding_table):
    pe = jnp.asarray(_PE_NP)
    mesh = plsc.VectorSubcoreMesh(
        core_axis_name="c", subcore_axis_name="s", num_cores=NC, num_subcores=NS
    )
    scratch = (
        [pltpu.VMEM((K, D_MODEL), jnp.float32) for _ in range(B * NBUF)]
        + [pltpu.VMEM((K, D_MODEL // 2), jnp.int32) for _ in range(NBUF)]
        + [pltpu.VMEM((B, POS_PER_W), jnp.int32)]
        + [pltpu.SemaphoreType.DMA for _ in range(2 * NBUF)]
    )
    f = pl.kernel(
        _sc_body,
        out_type=jax.ShapeDtypeStruct((B, S, D_MODEL), jnp.float32),
        mesh=mesh,
        scratch_types=scratch,
    )
    return f(tokens, pe, embedding_table)


def kernel(tokens, embedding_table):
    return _run(tokens, embedding_table)
